# baseline (speedup 1.0000x reference)
import functools

import jax
import jax.numpy as jnp
from jax import lax
from jax.experimental import pallas as pl
from jax.experimental.pallas import tpu as pltpu
from jax.experimental.pallas import tpu_sc as plsc

VOCAB = 1000000
EMBED = 16
BATCH = 16384
HIST = 200

TC_BN = 65536
TC_GRID = 16
NPAD = TC_BN * TC_GRID

NC = 2
NS = 16
NW = NC * NS
LANES = 16
ROWS_PER_W = BATCH // NW
GROUPS_PER_W = ROWS_PER_W // LANES
GLEN = HIST * LANES


def _tc_compress_body(w_ref, b_ref, embt_ref, t_ref):
    e = embt_ref[...]
    w = w_ref[...]
    t = jnp.dot(w, e, preferred_element_type=jnp.float32) + b_ref[0]
    t_ref[...] = t.reshape(TC_BN)


def _tc_compress(embt, wt, b):
    return pl.pallas_call(
        _tc_compress_body,
        grid=(TC_GRID,),
        in_specs=[
            pl.BlockSpec((1, EMBED), lambda i: (0, 0)),
            pl.BlockSpec(memory_space=pltpu.SMEM),
            pl.BlockSpec((EMBED, TC_BN), lambda i: (0, i)),
        ],
        out_specs=pl.BlockSpec((TC_BN,), lambda i: (i,)),
        out_shape=jax.ShapeDtypeStruct((NPAD,), jnp.float32),
    )(wt, b, embt)


def _sc_pool_body(t_hbm, xt_hbm, out_hbm, t_spmem, idx_v, vals_v, outw, gsem):
    c = lax.axis_index("c")
    s = lax.axis_index("s")
    wid = s * NC + c

    fill = NPAD // NS
    pltpu.sync_copy(t_hbm.at[pl.ds(s * fill, fill)],
                    t_spmem.at[pl.ds(s * fill, fill)])
    plsc.subcore_barrier()

    def group_body(g, carry):
        gid = wid * GROUPS_PER_W + g
        pltpu.sync_copy(xt_hbm.at[pl.ds(gid * GLEN, GLEN)], idx_v)
        pltpu.async_copy(t_spmem.at[idx_v], vals_v, gsem).wait()
        a0 = jnp.zeros((LANES,), jnp.float32)
        a1 = jnp.zeros((LANES,), jnp.float32)
        a2 = jnp.zeros((LANES,), jnp.float32)
        a3 = jnp.zeros((LANES,), jnp.float32)
        for j in range(0, HIST, 4):
            a0 = a0 + vals_v[pl.ds(j * LANES, LANES)]
            a1 = a1 + vals_v[pl.ds((j + 1) * LANES, LANES)]
            a2 = a2 + vals_v[pl.ds((j + 2) * LANES, LANES)]
            a3 = a3 + vals_v[pl.ds((j + 3) * LANES, LANES)]
        acc = (a0 + a1) + (a2 + a3)
        out16 = 1.0 / (1.0 + jnp.exp(-acc))
        outw[pl.ds(g * LANES, LANES)] = out16
        return carry

    lax.fori_loop(0, GROUPS_PER_W, group_body, 0)
    pltpu.sync_copy(outw, out_hbm.at[pl.ds(wid * ROWS_PER_W, ROWS_PER_W)])


_sc_pool = functools.partial(
    pl.kernel,
    out_type=jax.ShapeDtypeStruct((BATCH,), jnp.float32),
    mesh=plsc.VectorSubcoreMesh(core_axis_name="c", subcore_axis_name="s"),
    scratch_types=[
        pltpu.VMEM_SHARED((NPAD,), jnp.float32),
        pltpu.VMEM((GLEN,), jnp.int32),
        pltpu.VMEM((GLEN,), jnp.float32),
        pltpu.VMEM((ROWS_PER_W,), jnp.float32),
        pltpu.SemaphoreType.DMA,
    ],
)(_sc_pool_body)


def kernel(x, embedding, W, b):
    x = x.astype(jnp.int32)
    t = _tc_compress(embedding.T, W.reshape(1, EMBED) * (1.0 / HIST),
                     b * (1.0 / HIST))
    xt = (
        x.reshape(BATCH // LANES, LANES, HIST)
        .transpose(0, 2, 1)
        .reshape(BATCH // LANES * HIST * LANES)
    )
    out = _sc_pool(t, xt)
    return out.reshape(BATCH, 1)


# speedup vs baseline: 1.0018x; 1.0018x over previous
import functools

import jax
import jax.numpy as jnp
from jax import lax
from jax.experimental import pallas as pl
from jax.experimental.pallas import tpu as pltpu
from jax.experimental.pallas import tpu_sc as plsc

VOCAB = 1000000
EMBED = 16
BATCH = 16384
HIST = 200

TC_BN = 65536
TC_GRID = 16
NPAD = TC_BN * TC_GRID

NC = 2
NS = 16
NW = NC * NS
LANES = 16
ROWS_PER_W = BATCH // NW
GROUPS_PER_W = ROWS_PER_W // LANES
GLEN = HIST * LANES


def _tc_compress_body(w_ref, b_ref, embt_ref, t_ref):
    e = embt_ref[...]
    w = w_ref[...]
    t = jnp.dot(w, e, preferred_element_type=jnp.float32) + b_ref[0]
    t_ref[...] = t.reshape(TC_BN)


def _tc_compress(embt, wt, b):
    return pl.pallas_call(
        _tc_compress_body,
        grid=(TC_GRID,),
        in_specs=[
            pl.BlockSpec((1, EMBED), lambda i: (0, 0)),
            pl.BlockSpec(memory_space=pltpu.SMEM),
            pl.BlockSpec((EMBED, TC_BN), lambda i: (0, i)),
        ],
        out_specs=pl.BlockSpec((TC_BN,), lambda i: (i,)),
        out_shape=jax.ShapeDtypeStruct((NPAD,), jnp.float32),
    )(wt, b, embt)


def _sc_pool_body(t_hbm, xt_hbm, out_hbm, t_spmem, idx_v, vals_v, outw, gsem):
    c = lax.axis_index("c")
    s = lax.axis_index("s")
    wid = s * NC + c

    @pl.when(s == 0)
    def _fill():
        pltpu.sync_copy(t_hbm, t_spmem)

    plsc.subcore_barrier()

    def group_body(g, carry):
        gid = wid * GROUPS_PER_W + g
        pltpu.sync_copy(xt_hbm.at[pl.ds(gid * GLEN, GLEN)], idx_v)
        pltpu.async_copy(t_spmem.at[idx_v], vals_v, gsem).wait()
        a0 = jnp.zeros((LANES,), jnp.float32)
        a1 = jnp.zeros((LANES,), jnp.float32)
        a2 = jnp.zeros((LANES,), jnp.float32)
        a3 = jnp.zeros((LANES,), jnp.float32)
        for j in range(0, HIST, 4):
            a0 = a0 + vals_v[pl.ds(j * LANES, LANES)]
            a1 = a1 + vals_v[pl.ds((j + 1) * LANES, LANES)]
            a2 = a2 + vals_v[pl.ds((j + 2) * LANES, LANES)]
            a3 = a3 + vals_v[pl.ds((j + 3) * LANES, LANES)]
        acc = (a0 + a1) + (a2 + a3)
        out16 = 1.0 / (1.0 + jnp.exp(-acc))
        outw[pl.ds(g * LANES, LANES)] = out16
        return carry

    lax.fori_loop(0, GROUPS_PER_W, group_body, 0)
    pltpu.sync_copy(outw, out_hbm.at[pl.ds(wid * ROWS_PER_W, ROWS_PER_W)])


_sc_pool = functools.partial(
    pl.kernel,
    out_type=jax.ShapeDtypeStruct((BATCH,), jnp.float32),
    mesh=plsc.VectorSubcoreMesh(core_axis_name="c", subcore_axis_name="s"),
    scratch_types=[
        pltpu.VMEM_SHARED((NPAD,), jnp.float32),
        pltpu.VMEM((GLEN,), jnp.int32),
        pltpu.VMEM((GLEN,), jnp.float32),
        pltpu.VMEM((ROWS_PER_W,), jnp.float32),
        pltpu.SemaphoreType.DMA,
    ],
)(_sc_pool_body)


def kernel(x, embedding, W, b):
    x = x.astype(jnp.int32)
    t = _tc_compress(embedding.T, W.reshape(1, EMBED) * (1.0 / HIST),
                     b * (1.0 / HIST))
    xt = (
        x.reshape(BATCH // LANES, LANES, HIST)
        .transpose(0, 2, 1)
        .reshape(BATCH // LANES * HIST * LANES)
    )
    out = _sc_pool(t, xt)
    return out.reshape(BATCH, 1)


# speedup vs baseline: 1.3508x; 1.3483x over previous
import functools

import jax
import jax.numpy as jnp
from jax import lax
from jax.experimental import pallas as pl
from jax.experimental.pallas import tpu as pltpu
from jax.experimental.pallas import tpu_sc as plsc

VOCAB = 1000000
EMBED = 16
BATCH = 16384
HIST = 200

TC_BN = 65536
TC_GRID = 16
NPAD = TC_BN * TC_GRID

NC = 2
NS = 16
NW = NC * NS
LANES = 16
ROWS_PER_W = BATCH // NW
GROUPS_PER_W = ROWS_PER_W // LANES
GLEN = HIST * LANES


def _tc_compress_body(w_ref, b_ref, embt_ref, t_ref):
    e = embt_ref[...]
    w = w_ref[...]
    t = jnp.dot(w, e, preferred_element_type=jnp.float32) + b_ref[0]
    t_ref[...] = t.reshape(TC_BN)


def _tc_compress(embt, wt, b):
    return pl.pallas_call(
        _tc_compress_body,
        grid=(TC_GRID,),
        in_specs=[
            pl.BlockSpec((1, EMBED), lambda i: (0, 0)),
            pl.BlockSpec(memory_space=pltpu.SMEM),
            pl.BlockSpec((EMBED, TC_BN), lambda i: (0, i)),
        ],
        out_specs=pl.BlockSpec((TC_BN,), lambda i: (i,)),
        out_shape=jax.ShapeDtypeStruct((NPAD,), jnp.float32),
    )(wt, b, embt)


def _sc_pool_body(t_hbm, xt_hbm, out_hbm, t_spmem, idx_v, vals_v, outw, gsem):
    c = lax.axis_index("c")
    s = lax.axis_index("s")
    wid = s * NC + c

    @pl.when(s == 0)
    def _fill():
        pltpu.sync_copy(t_hbm, t_spmem)

    plsc.subcore_barrier()

    def group_body(g, carry):
        gid = wid * GROUPS_PER_W + g
        pltpu.sync_copy(xt_hbm.at[gid], idx_v)
        pltpu.async_copy(t_spmem.at[idx_v], vals_v, gsem).wait()
        a0 = jnp.zeros((LANES,), jnp.float32)
        a1 = jnp.zeros((LANES,), jnp.float32)
        a2 = jnp.zeros((LANES,), jnp.float32)
        a3 = jnp.zeros((LANES,), jnp.float32)
        for j in range(0, HIST, 4):
            a0 = a0 + vals_v[pl.ds(j * LANES, LANES)]
            a1 = a1 + vals_v[pl.ds((j + 1) * LANES, LANES)]
            a2 = a2 + vals_v[pl.ds((j + 2) * LANES, LANES)]
            a3 = a3 + vals_v[pl.ds((j + 3) * LANES, LANES)]
        acc = (a0 + a1) + (a2 + a3)
        out16 = 1.0 / (1.0 + jnp.exp(-acc))
        outw[pl.ds(g * LANES, LANES)] = out16
        return carry

    lax.fori_loop(0, GROUPS_PER_W, group_body, 0)
    pltpu.sync_copy(outw, out_hbm.at[pl.ds(wid * ROWS_PER_W, ROWS_PER_W)])


_sc_pool = functools.partial(
    pl.kernel,
    out_type=jax.ShapeDtypeStruct((BATCH,), jnp.float32),
    mesh=plsc.VectorSubcoreMesh(core_axis_name="c", subcore_axis_name="s"),
    scratch_types=[
        pltpu.VMEM_SHARED((NPAD,), jnp.float32),
        pltpu.VMEM((GLEN,), jnp.int32),
        pltpu.VMEM((GLEN,), jnp.float32),
        pltpu.VMEM((ROWS_PER_W,), jnp.float32),
        pltpu.SemaphoreType.DMA,
    ],
)(_sc_pool_body)


def kernel(x, embedding, W, b):
    x = x.astype(jnp.int32)
    t = _tc_compress(embedding.T, W.reshape(1, EMBED) * (1.0 / HIST),
                     b * (1.0 / HIST))
    xt = (
        x.reshape(BATCH // LANES, LANES, HIST)
        .transpose(0, 2, 1)
        .reshape(BATCH // LANES, HIST * LANES)
    )
    out = _sc_pool(t, xt)
    return out.reshape(BATCH, 1)
